# Initial kernel scaffold; baseline (speedup 1.0000x reference)
import jax
import jax.numpy as jnp
from jax.experimental import pallas as pl


def kernel(batch_proposals, batch_cls_prob):
    raise NotImplementedError("write your pallas kernel here")


# speedup vs baseline: 16.4840x; 16.4840x over previous
import jax
import jax.numpy as jnp
from jax.experimental import pallas as pl
from jax.experimental.pallas import tpu as pltpu

IMG_H, IMG_W = 800.0, 1333.0
PRE = 2000
POST = 1000
MIN_SIZE = 16.0
THR = 0.7
N = 20000
R, L = 256, 128
NPAD = R * L
M = 2048
BLK = 256
NBLK = M // BLK

_NEG_INF = float("-inf")


def _cmp_first(sa, ia, sb, ib):
    return (sa > sb) | ((sa == sb) & (ia < ib))


def _partner_rows(x, j_rows):
    g = R // (2 * j_rows)
    y = x.reshape(g, 2, j_rows, L)
    y = jnp.concatenate([y[:, 1:2], y[:, 0:1]], axis=1)
    return y.reshape(R, L)


def _to_col(row_slice, blk):
    b = jnp.broadcast_to(row_slice, (blk, blk))
    eye = jax.lax.broadcasted_iota(jnp.int32, (blk, blk), 0) == \
        jax.lax.broadcasted_iota(jnp.int32, (blk, blk), 1)
    return jnp.sum(jnp.where(eye, b, 0.0), axis=1, keepdims=True)


def _partner_lanes(x, j, lane_low):
    up = pltpu.roll(x, L - j, 1)
    dn = pltpu.roll(x, j, 1)
    return jnp.where(lane_low, up, dn)


def _proposal_kernel(s_ref, x1_ref, y1_ref, x2_ref, y2_ref, out_ref, s_scr,
                     oh_scr):
    f32 = jnp.float32
    bf16 = jnp.bfloat16

    x1 = jnp.clip(x1_ref[0], 0.0, IMG_W)
    y1 = jnp.clip(y1_ref[0], 0.0, IMG_H)
    x2 = jnp.clip(x2_ref[0], 0.0, IMG_W)
    y2 = jnp.clip(y2_ref[0], 0.0, IMG_H)
    valid = ((x2 - x1) >= MIN_SIZE) & ((y2 - y1) >= MIN_SIZE)
    s = jnp.where(valid, s_ref[0], _NEG_INF)

    row_i = jax.lax.broadcasted_iota(jnp.int32, (R, L), 0)
    lane_i = jax.lax.broadcasted_iota(jnp.int32, (R, L), 1)
    idx = row_i * L + lane_i

    arrs = [s, idx, x1, y1, x2, y2]
    k = 2
    while k <= NPAD:
        j = k // 2
        while j >= 1:
            if j >= L:
                jr = j // L
                low = (row_i & jr) == 0
                others = [_partner_rows(a, jr) for a in arrs]
            else:
                low = (lane_i & j) == 0
                others = [_partner_lanes(a, j, low) for a in arrs]
            if k >= L:
                dir_asc = (row_i & (k // L)) == 0
            else:
                dir_asc = (lane_i & k) == 0
            first = _cmp_first(arrs[0], arrs[1], others[0], others[1])
            keep_mine = first == (low == dir_asc)
            arrs = [jnp.where(keep_mine, a, o) for a, o in zip(arrs, others)]
            j //= 2
        k *= 2

    s16, _, x1t, y1t, x2t, y2t = [a[: M // L, :] for a in arrs]
    s_row = s16.reshape(1, M)
    x1_row = x1t.reshape(1, M)
    y1_row = y1t.reshape(1, M)
    x2_row = x2t.reshape(1, M)
    y2_row = y2t.reshape(1, M)

    iflat = jax.lax.broadcasted_iota(jnp.int32, (1, M), 1)
    live = iflat < PRE
    keep0 = (s_row > _NEG_INF) & live

    area_row = jnp.maximum(x2_row - x1_row, 0.0) * jnp.maximum(y2_row - y1_row, 0.0)
    for bj in range(NBLK):
        r0 = bj * BLK
        x1j = _to_col(x1_row[:, r0 : r0 + BLK], BLK)
        y1j = _to_col(y1_row[:, r0 : r0 + BLK], BLK)
        x2j = _to_col(x2_row[:, r0 : r0 + BLK], BLK)
        y2j = _to_col(y2_row[:, r0 : r0 + BLK], BLK)
        ltx = jnp.maximum(x1j, x1_row)
        lty = jnp.maximum(y1j, y1_row)
        rbx = jnp.minimum(x2j, x2_row)
        rby = jnp.minimum(y2j, y2_row)
        inter = jnp.maximum(rbx - ltx, 0.0) * jnp.maximum(rby - lty, 0.0)
        areaj = jnp.maximum(x2j - x1j, 0.0) * jnp.maximum(y2j - y1j, 0.0)
        union = areaj + area_row - inter
        iou = inter / jnp.maximum(union, 1e-9)
        jidx = jax.lax.broadcasted_iota(jnp.int32, (BLK, 1), 0) + r0
        s_blk = (iou > THR) & (jidx < iflat)
        s_scr[r0 : r0 + BLK, :] = s_blk.astype(bf16)

    supacc = jnp.zeros((1, M), f32)
    keep0_f = keep0.astype(f32)
    dec_blocks = []
    for b in range(NBLK):
        c0 = b * BLK
        init_b = keep0_f[:, c0 : c0 + BLK] * (supacc[:, c0 : c0 + BLK] == 0.0)
        sbb = s_scr[c0 : c0 + BLK, c0 : c0 + BLK]

        def cond(carry):
            return carry[1]

        def body(carry):
            a, _ = carry
            sup = jax.lax.dot_general(
                a.astype(bf16), sbb,
                (((1,), (0,)), ((), ())),
                preferred_element_type=f32,
            )
            a_new = init_b * (sup == 0.0)
            return a_new, jnp.any(a_new != a)

        dec_b, _ = jax.lax.while_loop(
            cond, body, (init_b, jnp.bool_(True))
        )
        sup_all = jax.lax.dot_general(
            dec_b.astype(bf16), s_scr[c0 : c0 + BLK, :],
            (((1,), (0,)), ((), ())),
            preferred_element_type=f32,
        )
        supacc = supacc + sup_all
        dec_blocks.append(dec_b)
    keep_f = jnp.concatenate(dec_blocks, axis=1)

    for bj in range(NBLK):
        r0 = bj * BLK
        jidx = jax.lax.broadcasted_iota(jnp.int32, (BLK, 1), 0) + r0
        tri = (jidx < iflat).astype(bf16)
        s_scr[r0 : r0 + BLK, :] = tri
    nk_f = live.astype(f32) * (1.0 - keep_f)
    cs_k = jax.lax.dot_general(
        keep_f.astype(bf16), s_scr[...],
        (((1,), (0,)), ((), ())), preferred_element_type=f32)
    cs_nk = jax.lax.dot_general(
        nk_f.astype(bf16), s_scr[...],
        (((1,), (0,)), ((), ())), preferred_element_type=f32)
    kt = jnp.sum(keep_f)
    dest = jnp.where(keep_f > 0.0, cs_k, kt + cs_nk)
    dest = jnp.where(live, dest, -1.0)

    p_iota = jax.lax.broadcasted_iota(jnp.int32, (1, 1024), 1).astype(f32)
    for bj in range(NBLK):
        r0 = bj * BLK
        dcol = _to_col(dest[:, r0 : r0 + BLK], BLK)
        oh_scr[r0 : r0 + BLK, :] = (dcol == p_iota).astype(f32)
    vrows = jnp.concatenate(
        [s_row, x1_row, y1_row, x2_row, y2_row, jnp.zeros((3, M), f32)], axis=0
    )
    out = jax.lax.dot_general(
        vrows, oh_scr[...], (((1,), (0,)), ((), ())),
        precision=jax.lax.Precision.HIGHEST,
        preferred_element_type=f32,
    )
    out_ref[0] = out


def kernel(batch_proposals, batch_cls_prob):
    b = batch_proposals.shape[0]
    scores = batch_cls_prob[:, :, 1]

    def prep(v, fill=0.0):
        v = jnp.pad(v, ((0, 0), (0, NPAD - N)), constant_values=fill)
        return v.reshape(b, R, L)

    s_in = prep(scores, _NEG_INF)
    x1_in = prep(batch_proposals[:, :, 0])
    y1_in = prep(batch_proposals[:, :, 1])
    x2_in = prep(batch_proposals[:, :, 2])
    y2_in = prep(batch_proposals[:, :, 3])

    spec = pl.BlockSpec((1, R, L), lambda i: (i, 0, 0))
    raw = pl.pallas_call(
        _proposal_kernel,
        grid=(b,),
        in_specs=[spec] * 5,
        out_specs=pl.BlockSpec((1, 8, 1024), lambda i: (i, 0, 0)),
        out_shape=jax.ShapeDtypeStruct((b, 8, 1024), jnp.float32),
        scratch_shapes=[pltpu.VMEM((M, M), jnp.bfloat16),
                        pltpu.VMEM((M, 1024), jnp.float32)],
        compiler_params=pltpu.CompilerParams(
            dimension_semantics=("parallel",),
        ),
    )(s_in, x1_in, y1_in, x2_in, y2_in)

    out_scores = raw[:, 0, :POST]
    out_boxes = jnp.stack(
        [raw[:, 1, :POST], raw[:, 2, :POST], raw[:, 3, :POST], raw[:, 4, :POST]],
        axis=-1,
    )
    return out_scores, out_boxes


# speedup vs baseline: 18.6227x; 1.1297x over previous
import jax
import jax.numpy as jnp
from jax.experimental import pallas as pl
from jax.experimental.pallas import tpu as pltpu

IMG_H, IMG_W = 800.0, 1333.0
PRE = 2000
POST = 1000
MIN_SIZE = 16.0
THR = 0.7
N = 20000
R, L = 256, 128
NPAD = R * L
M = 2048
BLK = 256
NBLK = M // BLK

_NEG_INF = float("-inf")


def _cmp_first(sa, ia, sb, ib):
    return (sa > sb) | ((sa == sb) & (ia < ib))


def _partner_rows(x, j_rows):
    g = R // (2 * j_rows)
    y = x.reshape(g, 2, j_rows, L)
    y = jnp.concatenate([y[:, 1:2], y[:, 0:1]], axis=1)
    return y.reshape(R, L)


def _to_col(row_slice, blk):
    b = jnp.broadcast_to(row_slice, (blk, blk))
    eye = jax.lax.broadcasted_iota(jnp.int32, (blk, blk), 0) == \
        jax.lax.broadcasted_iota(jnp.int32, (blk, blk), 1)
    return jnp.sum(jnp.where(eye, b, 0.0), axis=1, keepdims=True)


def _partner_lanes(x, j, lane_low):
    up = pltpu.roll(x, L - j, 1)
    dn = pltpu.roll(x, j, 1)
    return jnp.where(lane_low, up, dn)


def _proposal_kernel(s_ref, x1_ref, y1_ref, x2_ref, y2_ref, out_ref, s_scr,
                     oh_scr):
    f32 = jnp.float32
    bf16 = jnp.bfloat16

    x1 = jnp.clip(x1_ref[0], 0.0, IMG_W)
    y1 = jnp.clip(y1_ref[0], 0.0, IMG_H)
    x2 = jnp.clip(x2_ref[0], 0.0, IMG_W)
    y2 = jnp.clip(y2_ref[0], 0.0, IMG_H)
    valid = ((x2 - x1) >= MIN_SIZE) & ((y2 - y1) >= MIN_SIZE)
    s = jnp.where(valid, s_ref[0], _NEG_INF)

    row_i = jax.lax.broadcasted_iota(jnp.int32, (R, L), 0)
    lane_i = jax.lax.broadcasted_iota(jnp.int32, (R, L), 1)
    idx = row_i * L + lane_i

    arrs = [s, idx, x1, y1, x2, y2]
    k = 2
    while k <= NPAD:
        j = k // 2
        while j >= 1:
            if j < R:
                low = (row_i & j) == 0
                others = [_partner_rows(a, j) for a in arrs]
            else:
                jl = j // R
                low = (lane_i & jl) == 0
                others = [_partner_lanes(a, jl, low) for a in arrs]
            if k < R:
                dir_asc = (row_i & k) == 0
            else:
                dir_asc = (lane_i & (k // R)) == 0
            first = _cmp_first(arrs[0], arrs[1], others[0], others[1])
            keep_mine = first == (low == dir_asc)
            arrs = [jnp.where(keep_mine, a, o) for a, o in zip(arrs, others)]
            j //= 2
        k *= 2

    s16, _, x1t, y1t, x2t, y2t = [
        jnp.swapaxes(a[:, : M // R], 0, 1) for a in arrs
    ]
    s_row = s16.reshape(1, M)
    x1_row = x1t.reshape(1, M)
    y1_row = y1t.reshape(1, M)
    x2_row = x2t.reshape(1, M)
    y2_row = y2t.reshape(1, M)

    iflat = jax.lax.broadcasted_iota(jnp.int32, (1, M), 1)
    live = iflat < PRE
    keep0 = (s_row > _NEG_INF) & live

    area_row = jnp.maximum(x2_row - x1_row, 0.0) * jnp.maximum(y2_row - y1_row, 0.0)
    for bj in range(NBLK):
        r0 = bj * BLK
        x1j = _to_col(x1_row[:, r0 : r0 + BLK], BLK)
        y1j = _to_col(y1_row[:, r0 : r0 + BLK], BLK)
        x2j = _to_col(x2_row[:, r0 : r0 + BLK], BLK)
        y2j = _to_col(y2_row[:, r0 : r0 + BLK], BLK)
        ltx = jnp.maximum(x1j, x1_row)
        lty = jnp.maximum(y1j, y1_row)
        rbx = jnp.minimum(x2j, x2_row)
        rby = jnp.minimum(y2j, y2_row)
        inter = jnp.maximum(rbx - ltx, 0.0) * jnp.maximum(rby - lty, 0.0)
        areaj = jnp.maximum(x2j - x1j, 0.0) * jnp.maximum(y2j - y1j, 0.0)
        union = areaj + area_row - inter
        iou = inter / jnp.maximum(union, 1e-9)
        jidx = jax.lax.broadcasted_iota(jnp.int32, (BLK, 1), 0) + r0
        s_blk = (iou > THR) & (jidx < iflat)
        s_scr[r0 : r0 + BLK, :] = s_blk.astype(bf16)

    supacc = jnp.zeros((1, M), f32)
    keep0_f = keep0.astype(f32)
    dec_blocks = []
    for b in range(NBLK):
        c0 = b * BLK
        init_b = keep0_f[:, c0 : c0 + BLK] * (supacc[:, c0 : c0 + BLK] == 0.0)
        sbb = s_scr[c0 : c0 + BLK, c0 : c0 + BLK]

        def cond(carry):
            return carry[1]

        def body(carry):
            a, _ = carry
            sup = jax.lax.dot_general(
                a.astype(bf16), sbb,
                (((1,), (0,)), ((), ())),
                preferred_element_type=f32,
            )
            a_new = init_b * (sup == 0.0)
            return a_new, jnp.any(a_new != a)

        dec_b, _ = jax.lax.while_loop(
            cond, body, (init_b, jnp.bool_(True))
        )
        sup_all = jax.lax.dot_general(
            dec_b.astype(bf16), s_scr[c0 : c0 + BLK, :],
            (((1,), (0,)), ((), ())),
            preferred_element_type=f32,
        )
        supacc = supacc + sup_all
        dec_blocks.append(dec_b)
    keep_f = jnp.concatenate(dec_blocks, axis=1)

    for bj in range(NBLK):
        r0 = bj * BLK
        jidx = jax.lax.broadcasted_iota(jnp.int32, (BLK, 1), 0) + r0
        tri = (jidx < iflat).astype(bf16)
        s_scr[r0 : r0 + BLK, :] = tri
    nk_f = live.astype(f32) * (1.0 - keep_f)
    cs_k = jax.lax.dot_general(
        keep_f.astype(bf16), s_scr[...],
        (((1,), (0,)), ((), ())), preferred_element_type=f32)
    cs_nk = jax.lax.dot_general(
        nk_f.astype(bf16), s_scr[...],
        (((1,), (0,)), ((), ())), preferred_element_type=f32)
    kt = jnp.sum(keep_f)
    dest = jnp.where(keep_f > 0.0, cs_k, kt + cs_nk)
    dest = jnp.where(live, dest, -1.0)

    p_iota = jax.lax.broadcasted_iota(jnp.int32, (1, 1024), 1).astype(f32)
    for bj in range(NBLK):
        r0 = bj * BLK
        dcol = _to_col(dest[:, r0 : r0 + BLK], BLK)
        oh_scr[r0 : r0 + BLK, :] = (dcol == p_iota).astype(f32)
    vrows = jnp.concatenate(
        [s_row, x1_row, y1_row, x2_row, y2_row, jnp.zeros((3, M), f32)], axis=0
    )
    out = jax.lax.dot_general(
        vrows, oh_scr[...], (((1,), (0,)), ((), ())),
        precision=jax.lax.Precision.HIGHEST,
        preferred_element_type=f32,
    )
    out_ref[0] = out


def kernel(batch_proposals, batch_cls_prob):
    b = batch_proposals.shape[0]
    scores = batch_cls_prob[:, :, 1]

    def prep(v, fill=0.0):
        v = jnp.pad(v, ((0, 0), (0, NPAD - N)), constant_values=fill)
        return v.reshape(b, R, L)

    s_in = prep(scores, _NEG_INF)
    x1_in = prep(batch_proposals[:, :, 0])
    y1_in = prep(batch_proposals[:, :, 1])
    x2_in = prep(batch_proposals[:, :, 2])
    y2_in = prep(batch_proposals[:, :, 3])

    spec = pl.BlockSpec((1, R, L), lambda i: (i, 0, 0))
    raw = pl.pallas_call(
        _proposal_kernel,
        grid=(b,),
        in_specs=[spec] * 5,
        out_specs=pl.BlockSpec((1, 8, 1024), lambda i: (i, 0, 0)),
        out_shape=jax.ShapeDtypeStruct((b, 8, 1024), jnp.float32),
        scratch_shapes=[pltpu.VMEM((M, M), jnp.bfloat16),
                        pltpu.VMEM((M, 1024), jnp.float32)],
        compiler_params=pltpu.CompilerParams(
            dimension_semantics=("parallel",),
        ),
    )(s_in, x1_in, y1_in, x2_in, y2_in)

    out_scores = raw[:, 0, :POST]
    out_boxes = jnp.stack(
        [raw[:, 1, :POST], raw[:, 2, :POST], raw[:, 3, :POST], raw[:, 4, :POST]],
        axis=-1,
    )
    return out_scores, out_boxes


# speedup vs baseline: 21.7827x; 1.1697x over previous
import jax
import jax.numpy as jnp
from jax.experimental import pallas as pl
from jax.experimental.pallas import tpu as pltpu
from jax.experimental.pallas import tpu_sc as plsc

IMG_H, IMG_W = 800.0, 1333.0
PRE = 2000
POST = 1000
MIN_SIZE = 16.0
THR = 0.7
N = 20000
R, L = 256, 128
NPAD = R * L
M = 2048
BLK = 256
NBLK = M // BLK
GW = 128

_NEG_INF = float("-inf")


def _cmp_first(sa, ia, sb, ib):
    return (sa > sb) | ((sa == sb) & (ia < ib))


def _partner_rows(x, j_rows):
    g = R // (2 * j_rows)
    y = x.reshape(g, 2, j_rows, L)
    y = jnp.concatenate([y[:, 1:2], y[:, 0:1]], axis=1)
    return y.reshape(R, L)


def _to_col(row_slice, blk):
    b = jnp.broadcast_to(row_slice, (blk, blk))
    eye = jax.lax.broadcasted_iota(jnp.int32, (blk, blk), 0) == \
        jax.lax.broadcasted_iota(jnp.int32, (blk, blk), 1)
    return jnp.sum(jnp.where(eye, b, 0.0), axis=1, keepdims=True)


def _partner_lanes(x, j, lane_low):
    up = pltpu.roll(x, L - j, 1)
    dn = pltpu.roll(x, j, 1)
    return jnp.where(lane_low, up, dn)


def _topk_kernel(s_ref, x1_ref, y1_ref, x2_ref, y2_ref, s_out, i_out):
    x1 = jnp.clip(x1_ref[0], 0.0, IMG_W)
    y1 = jnp.clip(y1_ref[0], 0.0, IMG_H)
    x2 = jnp.clip(x2_ref[0], 0.0, IMG_W)
    y2 = jnp.clip(y2_ref[0], 0.0, IMG_H)
    valid = ((x2 - x1) >= MIN_SIZE) & ((y2 - y1) >= MIN_SIZE)
    s = jnp.where(valid, s_ref[0], _NEG_INF)

    row_i = jax.lax.broadcasted_iota(jnp.int32, (R, L), 0)
    lane_i = jax.lax.broadcasted_iota(jnp.int32, (R, L), 1)
    idx = row_i * L + lane_i

    arrs = [s, idx]
    k = 2
    while k <= NPAD:
        j = k // 2
        while j >= 1:
            if j < R:
                low = (row_i & j) == 0
                others = [_partner_rows(a, j) for a in arrs]
            else:
                jl = j // R
                low = (lane_i & jl) == 0
                others = [_partner_lanes(a, jl, low) for a in arrs]
            if k < R:
                dir_asc = (row_i & k) == 0
            else:
                dir_asc = (lane_i & (k // R)) == 0
            first = _cmp_first(arrs[0], arrs[1], others[0], others[1])
            keep_mine = first == (low == dir_asc)
            arrs = [jnp.where(keep_mine, a, o) for a, o in zip(arrs, others)]
            j //= 2
        k *= 2

    s_out[0] = jnp.swapaxes(arrs[0][:, : M // R], 0, 1).reshape(1, M)
    i_out[0] = jnp.swapaxes(arrs[1][:, : M // R], 0, 1).reshape(1, M)


def _sc_gather(table, indices):
    mesh = plsc.VectorSubcoreMesh(core_axis_name="c", subcore_axis_name="s")
    num_idx = indices.shape[1]

    @pl.kernel(
        out_type=jax.ShapeDtypeStruct((num_idx, 128), table.dtype), mesh=mesh
    )
    def kern(t_hbm, i_hbm, o_hbm):
        def body(i_vmem, o_vmem):
            pltpu.sync_copy(t_hbm.at[i_vmem.at[0]], o_vmem)

        pltpu.emit_pipeline(
            body,
            grid=(num_idx // GW,),
            in_specs=[pl.BlockSpec((1, GW), index_map=lambda i: (0, i))],
            out_specs=[pl.BlockSpec((GW, 128), index_map=lambda i: (i, 0))],
            core_axis_name=("c", "s"),
            dimension_semantics=(pltpu.PARALLEL,),
        )(i_hbm, o_hbm)

    return kern(table, indices)


def _nms_kernel(s_ref, c_ref, out_ref, s_scr, oh_scr):
    f32 = jnp.float32
    bf16 = jnp.bfloat16

    s_row = s_ref[0]
    coords = c_ref[0]
    x1_row = jnp.clip(coords[0:1], 0.0, IMG_W)
    y1_row = jnp.clip(coords[1:2], 0.0, IMG_H)
    x2_row = jnp.clip(coords[2:3], 0.0, IMG_W)
    y2_row = jnp.clip(coords[3:4], 0.0, IMG_H)

    iflat = jax.lax.broadcasted_iota(jnp.int32, (1, M), 1)
    live = iflat < PRE
    keep0 = (s_row > _NEG_INF) & live

    area_row = jnp.maximum(x2_row - x1_row, 0.0) * jnp.maximum(y2_row - y1_row, 0.0)
    for bj in range(NBLK):
        r0 = bj * BLK
        x1j = _to_col(x1_row[:, r0 : r0 + BLK], BLK)
        y1j = _to_col(y1_row[:, r0 : r0 + BLK], BLK)
        x2j = _to_col(x2_row[:, r0 : r0 + BLK], BLK)
        y2j = _to_col(y2_row[:, r0 : r0 + BLK], BLK)
        ltx = jnp.maximum(x1j, x1_row)
        lty = jnp.maximum(y1j, y1_row)
        rbx = jnp.minimum(x2j, x2_row)
        rby = jnp.minimum(y2j, y2_row)
        inter = jnp.maximum(rbx - ltx, 0.0) * jnp.maximum(rby - lty, 0.0)
        areaj = jnp.maximum(x2j - x1j, 0.0) * jnp.maximum(y2j - y1j, 0.0)
        union = areaj + area_row - inter
        iou = inter / jnp.maximum(union, 1e-9)
        jidx = jax.lax.broadcasted_iota(jnp.int32, (BLK, 1), 0) + r0
        s_blk = (iou > THR) & (jidx < iflat)
        s_scr[r0 : r0 + BLK, :] = s_blk.astype(bf16)

    supacc = jnp.zeros((1, M), f32)
    keep0_f = keep0.astype(f32)
    dec_blocks = []
    for b in range(NBLK):
        c0 = b * BLK
        init_b = keep0_f[:, c0 : c0 + BLK] * (supacc[:, c0 : c0 + BLK] == 0.0)
        sbb = s_scr[c0 : c0 + BLK, c0 : c0 + BLK]

        def cond(carry):
            return carry[1]

        def body(carry):
            a, _ = carry
            sup = jax.lax.dot_general(
                a.astype(bf16), sbb,
                (((1,), (0,)), ((), ())),
                preferred_element_type=f32,
            )
            a_new = init_b * (sup == 0.0)
            return a_new, jnp.any(a_new != a)

        dec_b, _ = jax.lax.while_loop(cond, body, (init_b, jnp.bool_(True)))
        sup_all = jax.lax.dot_general(
            dec_b.astype(bf16), s_scr[c0 : c0 + BLK, :],
            (((1,), (0,)), ((), ())),
            preferred_element_type=f32,
        )
        supacc = supacc + sup_all
        dec_blocks.append(dec_b)
    keep_f = jnp.concatenate(dec_blocks, axis=1)

    for bj in range(NBLK):
        r0 = bj * BLK
        jidx = jax.lax.broadcasted_iota(jnp.int32, (BLK, 1), 0) + r0
        tri = (jidx < iflat).astype(bf16)
        s_scr[r0 : r0 + BLK, :] = tri
    nk_f = live.astype(f32) * (1.0 - keep_f)
    cs_k = jax.lax.dot_general(
        keep_f.astype(bf16), s_scr[...],
        (((1,), (0,)), ((), ())), preferred_element_type=f32)
    cs_nk = jax.lax.dot_general(
        nk_f.astype(bf16), s_scr[...],
        (((1,), (0,)), ((), ())), preferred_element_type=f32)
    kt = jnp.sum(keep_f)
    dest = jnp.where(keep_f > 0.0, cs_k, kt + cs_nk)
    dest = jnp.where(live, dest, -1.0)

    p_iota = jax.lax.broadcasted_iota(jnp.int32, (1, 1024), 1).astype(f32)
    for bj in range(NBLK):
        r0 = bj * BLK
        dcol = _to_col(dest[:, r0 : r0 + BLK], BLK)
        oh_scr[r0 : r0 + BLK, :] = (dcol == p_iota).astype(f32)
    vrows = jnp.concatenate(
        [s_row, x1_row, y1_row, x2_row, y2_row, jnp.zeros((3, M), f32)], axis=0
    )
    out = jax.lax.dot_general(
        vrows, oh_scr[...], (((1,), (0,)), ((), ())),
        precision=jax.lax.Precision.HIGHEST,
        preferred_element_type=f32,
    )
    out_ref[0] = out


def kernel(batch_proposals, batch_cls_prob):
    b = batch_proposals.shape[0]
    scores = batch_cls_prob[:, :, 1]

    def prep(v, fill=0.0):
        v = jnp.pad(v, ((0, 0), (0, NPAD - N)), constant_values=fill)
        return v.reshape(b, R, L)

    s_in = prep(scores, _NEG_INF)
    x1_in = prep(batch_proposals[:, :, 0])
    y1_in = prep(batch_proposals[:, :, 1])
    x2_in = prep(batch_proposals[:, :, 2])
    y2_in = prep(batch_proposals[:, :, 3])

    spec = pl.BlockSpec((1, R, L), lambda i: (i, 0, 0))
    ospec = pl.BlockSpec((1, 1, M), lambda i: (i, 0, 0))
    s_top, i_top = pl.pallas_call(
        _topk_kernel,
        grid=(b,),
        in_specs=[spec] * 5,
        out_specs=[ospec, ospec],
        out_shape=[
            jax.ShapeDtypeStruct((b, 1, M), jnp.float32),
            jax.ShapeDtypeStruct((b, 1, M), jnp.int32),
        ],
        compiler_params=pltpu.CompilerParams(
            dimension_semantics=("parallel",),
        ),
    )(s_in, x1_in, y1_in, x2_in, y2_in)

    rows = jnp.pad(batch_proposals, ((0, 0), (0, NPAD - N), (0, 124)))
    table = rows.reshape(b * NPAD, 128)
    base = (jnp.arange(b, dtype=jnp.int32) * NPAD).reshape(b, 1)
    flat_idx = (i_top.reshape(b, M) + base).reshape(1, b * M)
    gathered = _sc_gather(table, flat_idx)
    coords = jnp.transpose(gathered[:, :4].reshape(b, M, 4), (0, 2, 1))

    raw = pl.pallas_call(
        _nms_kernel,
        grid=(b,),
        in_specs=[
            pl.BlockSpec((1, 1, M), lambda i: (i, 0, 0)),
            pl.BlockSpec((1, 4, M), lambda i: (i, 0, 0)),
        ],
        out_specs=pl.BlockSpec((1, 8, 1024), lambda i: (i, 0, 0)),
        out_shape=jax.ShapeDtypeStruct((b, 8, 1024), jnp.float32),
        scratch_shapes=[pltpu.VMEM((M, M), jnp.bfloat16),
                        pltpu.VMEM((M, 1024), jnp.float32)],
        compiler_params=pltpu.CompilerParams(
            dimension_semantics=("parallel",),
        ),
    )(s_top, coords)

    out_scores = raw[:, 0, :POST]
    out_boxes = jnp.stack(
        [raw[:, 1, :POST], raw[:, 2, :POST], raw[:, 3, :POST], raw[:, 4, :POST]],
        axis=-1,
    )
    return out_scores, out_boxes


# speedup vs baseline: 22.2260x; 1.0204x over previous
import jax
import jax.numpy as jnp
from jax.experimental import pallas as pl
from jax.experimental.pallas import tpu as pltpu
from jax.experimental.pallas import tpu_sc as plsc

IMG_H, IMG_W = 800.0, 1333.0
PRE = 2000
POST = 1000
MIN_SIZE = 16.0
THR = 0.7
N = 20000
R, L = 256, 128
NPAD = R * L
M = 2048
BLK = 256
NBLK = M // BLK
GW = 128

_NEG_INF = float("-inf")


def _cmp_first(sa, ia, sb, ib):
    return (sa > sb) | ((sa == sb) & (ia < ib))


def _partner_rows(x, j_rows):
    g = R // (2 * j_rows)
    y = x.reshape(g, 2, j_rows, L)
    y = jnp.concatenate([y[:, 1:2], y[:, 0:1]], axis=1)
    return y.reshape(R, L)


def _to_col(row_slice, blk):
    b = jnp.broadcast_to(row_slice, (blk, blk))
    eye = jax.lax.broadcasted_iota(jnp.int32, (blk, blk), 0) == \
        jax.lax.broadcasted_iota(jnp.int32, (blk, blk), 1)
    return jnp.sum(jnp.where(eye, b, 0.0), axis=1, keepdims=True)


def _partner_lanes(x, j, lane_low):
    up = pltpu.roll(x, L - j, 1)
    dn = pltpu.roll(x, j, 1)
    return jnp.where(lane_low, up, dn)


def _topk_kernel(s_ref, x1_ref, y1_ref, x2_ref, y2_ref, s_out, i_out):
    x1 = jnp.clip(x1_ref[0], 0.0, IMG_W)
    y1 = jnp.clip(y1_ref[0], 0.0, IMG_H)
    x2 = jnp.clip(x2_ref[0], 0.0, IMG_W)
    y2 = jnp.clip(y2_ref[0], 0.0, IMG_H)
    valid = ((x2 - x1) >= MIN_SIZE) & ((y2 - y1) >= MIN_SIZE)
    s = jnp.where(valid, s_ref[0], _NEG_INF)

    row_i = jax.lax.broadcasted_iota(jnp.int32, (R, L), 0)
    lane_i = jax.lax.broadcasted_iota(jnp.int32, (R, L), 1)
    idx = row_i * L + lane_i

    arrs = [s, idx]
    k = 2
    while k <= NPAD:
        j = k // 2
        while j >= 1:
            if j < R:
                low = (row_i & j) == 0
                others = [_partner_rows(a, j) for a in arrs]
            else:
                jl = j // R
                low = (lane_i & jl) == 0
                others = [_partner_lanes(a, jl, low) for a in arrs]
            if k < R:
                dir_asc = (row_i & k) == 0
            else:
                dir_asc = (lane_i & (k // R)) == 0
            first = _cmp_first(arrs[0], arrs[1], others[0], others[1])
            keep_mine = first == (low == dir_asc)
            arrs = [jnp.where(keep_mine, a, o) for a, o in zip(arrs, others)]
            j //= 2
        k *= 2

    s_out[0] = jnp.swapaxes(arrs[0][:, : M // R], 0, 1).reshape(1, M)
    i_out[0] = jnp.swapaxes(arrs[1][:, : M // R], 0, 1).reshape(1, M)


def _sc_gather(table, indices):
    mesh = plsc.VectorSubcoreMesh(core_axis_name="c", subcore_axis_name="s")
    num_idx = indices.shape[1]

    @pl.kernel(
        out_type=jax.ShapeDtypeStruct((num_idx, 128), table.dtype), mesh=mesh
    )
    def kern(t_hbm, i_hbm, o_hbm):
        def body(i_vmem, o_vmem):
            pltpu.sync_copy(t_hbm.at[i_vmem.at[0]], o_vmem)

        pltpu.emit_pipeline(
            body,
            grid=(num_idx // GW,),
            in_specs=[pl.BlockSpec((1, GW), index_map=lambda i: (0, i))],
            out_specs=[pl.BlockSpec((GW, 128), index_map=lambda i: (i, 0))],
            core_axis_name=("c", "s"),
            dimension_semantics=(pltpu.PARALLEL,),
        )(i_hbm, o_hbm)

    return kern(table, indices)


def _nms_kernel(s_ref, c_ref, out_ref, s_scr, oh_scr):
    f32 = jnp.float32
    bf16 = jnp.bfloat16

    s_row = s_ref[0]
    coords = c_ref[0]
    x1_row = jnp.clip(coords[0:1], 0.0, IMG_W)
    y1_row = jnp.clip(coords[1:2], 0.0, IMG_H)
    x2_row = jnp.clip(coords[2:3], 0.0, IMG_W)
    y2_row = jnp.clip(coords[3:4], 0.0, IMG_H)

    iflat = jax.lax.broadcasted_iota(jnp.int32, (1, M), 1)
    live = iflat < PRE
    keep0 = (s_row > _NEG_INF) & live

    area_row = jnp.maximum(x2_row - x1_row, 0.0) * jnp.maximum(y2_row - y1_row, 0.0)
    for bj in range(NBLK):
        r0 = bj * BLK
        w = M - r0
        x1j = _to_col(x1_row[:, r0 : r0 + BLK], BLK)
        y1j = _to_col(y1_row[:, r0 : r0 + BLK], BLK)
        x2j = _to_col(x2_row[:, r0 : r0 + BLK], BLK)
        y2j = _to_col(y2_row[:, r0 : r0 + BLK], BLK)
        ltx = jnp.maximum(x1j, x1_row[:, r0:])
        lty = jnp.maximum(y1j, y1_row[:, r0:])
        rbx = jnp.minimum(x2j, x2_row[:, r0:])
        rby = jnp.minimum(y2j, y2_row[:, r0:])
        inter = jnp.maximum(rbx - ltx, 0.0) * jnp.maximum(rby - lty, 0.0)
        areaj = jnp.maximum(x2j - x1j, 0.0) * jnp.maximum(y2j - y1j, 0.0)
        union = areaj + area_row[:, r0:] - inter
        iou = inter / jnp.maximum(union, 1e-9)
        jidx = jax.lax.broadcasted_iota(jnp.int32, (BLK, 1), 0) + r0
        s_blk = (iou > THR) & (jidx < iflat[:, r0:])
        if r0 > 0:
            s_scr[r0 : r0 + BLK, 0:r0] = jnp.zeros((BLK, r0), bf16)
        s_scr[r0 : r0 + BLK, r0:] = s_blk.astype(bf16)

    supacc = jnp.zeros((1, M), f32)
    keep0_f = keep0.astype(f32)
    dec_blocks = []
    for b in range(NBLK):
        c0 = b * BLK
        init_b = keep0_f[:, c0 : c0 + BLK] * (supacc[:, c0 : c0 + BLK] == 0.0)
        sbb = s_scr[c0 : c0 + BLK, c0 : c0 + BLK]

        def cond(carry):
            return carry[1]

        def body(carry):
            a, _ = carry
            sup = jax.lax.dot_general(
                a.astype(bf16), sbb,
                (((1,), (0,)), ((), ())),
                preferred_element_type=f32,
            )
            a_new = init_b * (sup == 0.0)
            return a_new, jnp.any(a_new != a)

        dec_b, _ = jax.lax.while_loop(cond, body, (init_b, jnp.bool_(True)))
        sup_all = jax.lax.dot_general(
            dec_b.astype(bf16), s_scr[c0 : c0 + BLK, :],
            (((1,), (0,)), ((), ())),
            preferred_element_type=f32,
        )
        supacc = supacc + sup_all
        dec_blocks.append(dec_b)
    keep_f = jnp.concatenate(dec_blocks, axis=1)

    for bj in range(NBLK):
        r0 = bj * BLK
        jidx = jax.lax.broadcasted_iota(jnp.int32, (BLK, 1), 0) + r0
        tri = (jidx < iflat).astype(bf16)
        s_scr[r0 : r0 + BLK, :] = tri
    nk_f = live.astype(f32) * (1.0 - keep_f)
    cs_k = jax.lax.dot_general(
        keep_f.astype(bf16), s_scr[...],
        (((1,), (0,)), ((), ())), preferred_element_type=f32)
    cs_nk = jax.lax.dot_general(
        nk_f.astype(bf16), s_scr[...],
        (((1,), (0,)), ((), ())), preferred_element_type=f32)
    kt = jnp.sum(keep_f)
    dest = jnp.where(keep_f > 0.0, cs_k, kt + cs_nk)
    dest = jnp.where(live, dest, -1.0)

    p_iota = jax.lax.broadcasted_iota(jnp.int32, (1, 1024), 1).astype(f32)
    for bj in range(NBLK):
        r0 = bj * BLK
        dcol = _to_col(dest[:, r0 : r0 + BLK], BLK)
        oh_scr[r0 : r0 + BLK, :] = (dcol == p_iota).astype(f32)
    vrows = jnp.concatenate(
        [s_row, x1_row, y1_row, x2_row, y2_row, jnp.zeros((3, M), f32)], axis=0
    )
    out = jax.lax.dot_general(
        vrows, oh_scr[...], (((1,), (0,)), ((), ())),
        precision=jax.lax.Precision.HIGHEST,
        preferred_element_type=f32,
    )
    out_ref[0] = out


def kernel(batch_proposals, batch_cls_prob):
    b = batch_proposals.shape[0]
    scores = batch_cls_prob[:, :, 1]

    def prep(v, fill=0.0):
        v = jnp.pad(v, ((0, 0), (0, NPAD - N)), constant_values=fill)
        return v.reshape(b, R, L)

    s_in = prep(scores, _NEG_INF)
    x1_in = prep(batch_proposals[:, :, 0])
    y1_in = prep(batch_proposals[:, :, 1])
    x2_in = prep(batch_proposals[:, :, 2])
    y2_in = prep(batch_proposals[:, :, 3])

    spec = pl.BlockSpec((1, R, L), lambda i: (i, 0, 0))
    ospec = pl.BlockSpec((1, 1, M), lambda i: (i, 0, 0))
    s_top, i_top = pl.pallas_call(
        _topk_kernel,
        grid=(b,),
        in_specs=[spec] * 5,
        out_specs=[ospec, ospec],
        out_shape=[
            jax.ShapeDtypeStruct((b, 1, M), jnp.float32),
            jax.ShapeDtypeStruct((b, 1, M), jnp.int32),
        ],
        compiler_params=pltpu.CompilerParams(
            dimension_semantics=("parallel",),
        ),
    )(s_in, x1_in, y1_in, x2_in, y2_in)

    rows = jnp.pad(batch_proposals, ((0, 0), (0, 0), (0, 124)))
    table = rows.reshape(b * N, 128)
    base = (jnp.arange(b, dtype=jnp.int32) * N).reshape(b, 1)
    flat_idx = (i_top.reshape(b, M) + base).reshape(1, b * M)
    gathered = _sc_gather(table, flat_idx)
    coords = jnp.transpose(gathered[:, :4].reshape(b, M, 4), (0, 2, 1))

    raw = pl.pallas_call(
        _nms_kernel,
        grid=(b,),
        in_specs=[
            pl.BlockSpec((1, 1, M), lambda i: (i, 0, 0)),
            pl.BlockSpec((1, 4, M), lambda i: (i, 0, 0)),
        ],
        out_specs=pl.BlockSpec((1, 8, 1024), lambda i: (i, 0, 0)),
        out_shape=jax.ShapeDtypeStruct((b, 8, 1024), jnp.float32),
        scratch_shapes=[pltpu.VMEM((M, M), jnp.bfloat16),
                        pltpu.VMEM((M, 1024), jnp.float32)],
        compiler_params=pltpu.CompilerParams(
            dimension_semantics=("parallel",),
        ),
    )(s_top, coords)

    out_scores = raw[:, 0, :POST]
    out_boxes = jnp.stack(
        [raw[:, 1, :POST], raw[:, 2, :POST], raw[:, 3, :POST], raw[:, 4, :POST]],
        axis=-1,
    )
    return out_scores, out_boxes
